# Initial kernel scaffold; baseline (speedup 1.0000x reference)
import jax
import jax.numpy as jnp
from jax.experimental import pallas as pl


def kernel(inputs, token_table, position_table):
    raise NotImplementedError("write your pallas kernel here")


# speedup vs baseline: 4.8408x; 4.8408x over previous
import jax
import jax.numpy as jnp
from jax import lax
from jax.experimental import pallas as pl
from jax.experimental.pallas import tpu as pltpu, tpu_sc as plsc

_NC = 2
_NS = 16
_NW = _NC * _NS

_B = 16384
_L = 200
_D = 32
_N = _B * _L
_R = _N // _NW
_S = 100
_G = 8
_C = _S * _G
_SEQ_PER_CHUNK = _C // _L
_NCHUNK = _R // _C


def _emb_kernel(tok_hbm, idx_hbm, pos_hbm, out_hbm,
                pos_v, idx_v, data_v, gsem0, gsem1, ssem0, ssem1):
  gsems = (gsem0, gsem1)
  ssems = (ssem0, ssem1)
  wid = lax.axis_index("s") * _NC + lax.axis_index("c")
  base = wid * _R
  idx_row0 = wid * (_R // _S)

  def fire_gathers(c, p):
    pltpu.sync_copy(idx_hbm.at[pl.ds(idx_row0 + c * _G, _G), :], idx_v.at[p])
    for j in range(_G):
      pltpu.async_copy(
          tok_hbm.at[idx_v.at[p, j]],
          data_v.at[p, pl.ds(j * _S, _S), :],
          gsems[p],
      )

  def drain_gathers(p):
    pltpu.make_async_copy(
        out_hbm.at[pl.ds(0, _C), :], data_v.at[p], gsems[p]).wait()

  def wait_store(p):
    pltpu.make_async_copy(
        data_v.at[p], out_hbm.at[pl.ds(0, _C), :], ssems[p]).wait()

  pltpu.sync_copy(pos_hbm, pos_v)
  fire_gathers(0, 0)

  @pl.loop(0, _NCHUNK, step=2)
  def _chunks(g):
    for p in range(2):
      c = g + p
      @pl.when(c >= 1)
      def _():
        wait_store(1 - p)

      @pl.when(c + 1 < _NCHUNK)
      def _():
        fire_gathers(c + 1, 1 - p)

      drain_gathers(p)

      @pl.loop(0, _L)
      def _add(l):
        ph0 = pos_v[l, pl.ds(0, 16)]
        ph1 = pos_v[l, pl.ds(16, 16)]
        for s in range(_SEQ_PER_CHUNK):
          row = s * _L + l
          plsc.addupdate(data_v.at[p, row, pl.ds(0, 16)], ph0)
          plsc.addupdate(data_v.at[p, row, pl.ds(16, 16)], ph1)

      pltpu.async_copy(
          data_v.at[p],
          out_hbm.at[pl.ds(base + c * _C, _C), :],
          ssems[p],
      )

  wait_store((_NCHUNK - 1) % 2)


def kernel(inputs, token_table, position_table):
  idx2d = inputs.astype(jnp.int32).reshape(_N // _S, _S)
  run = pl.kernel(
      _emb_kernel,
      out_type=jax.ShapeDtypeStruct((_N, _D), jnp.float32),
      mesh=plsc.VectorSubcoreMesh(core_axis_name="c", subcore_axis_name="s"),
      compiler_params=pltpu.CompilerParams(use_tc_tiling_on_sc=False),
      scratch_types=[
          pltpu.VMEM((_L, _D), jnp.float32),
          pltpu.VMEM((2, _G, _S), jnp.int32),
          pltpu.VMEM((2, _C, _D), jnp.float32),
          pltpu.SemaphoreType.DMA,
          pltpu.SemaphoreType.DMA,
          pltpu.SemaphoreType.DMA,
          pltpu.SemaphoreType.DMA,
      ],
  )
  out = run(token_table, idx2d, position_table)
  return out.reshape(_B, _L, _D)
